# Initial kernel scaffold; baseline (speedup 1.0000x reference)
import jax
import jax.numpy as jnp
from jax.experimental import pallas as pl


def kernel(x, mask):
    raise NotImplementedError("write your pallas kernel here")


# speedup vs baseline: 1.5283x; 1.5283x over previous
import functools

import jax
import jax.numpy as jnp
from jax import lax
from jax.experimental import pallas as pl
from jax.experimental.pallas import tpu as pltpu
from jax.experimental.pallas import tpu_sc as plsc

ROWS, COLS = 4096, 8192
LANES = 16
CHUNKS = COLS // LANES
R = 4

_info = plsc.get_sparse_core_info()
NC, NS = _info.num_cores, _info.num_subcores
NW = NC * NS
ROWS_PER_W = ROWS // NW
GROUPS = ROWS_PER_W // R


def _body(x_hbm, m_hbm, out_hbm, xv, mv, ov):
    wid = lax.axis_index("s") * NC + lax.axis_index("c")
    base = wid * ROWS_PER_W

    def group(g, _):
        row0 = base + g * R
        pltpu.sync_copy(x_hbm.at[pl.ds(row0, R)], xv)
        pltpu.sync_copy(m_hbm.at[pl.ds(row0, R)], mv)

        def chunk(j, carries):
            off = j * LANES
            new = []
            for r in range(R):
                v = xv[r, pl.ds(off, LANES)] * mv[r, pl.ds(off, LANES)]
                s = plsc.cumsum(v) + carries[r]
                ov[r, pl.ds(off, LANES)] = s
                new.append(s[LANES - 1])
            return tuple(new)

        lax.fori_loop(0, CHUNKS, chunk, (jnp.float32(0),) * R, unroll=False)
        pltpu.sync_copy(ov, out_hbm.at[pl.ds(row0, R)])
        return _

    lax.fori_loop(0, GROUPS, group, 0, unroll=False)


@jax.jit
def _masked_cumsum(x, mask_f):
    mesh = plsc.VectorSubcoreMesh(core_axis_name="c", subcore_axis_name="s")
    return pl.kernel(
        _body,
        out_type=jax.ShapeDtypeStruct((ROWS, COLS), jnp.float32),
        mesh=mesh,
        scratch_types=[
            pltpu.VMEM((R, COLS), jnp.float32),
            pltpu.VMEM((R, COLS), jnp.float32),
            pltpu.VMEM((R, COLS), jnp.float32),
        ],
        compiler_params=pltpu.CompilerParams(needs_layout_passes=False),
    )(x, mask_f)


def kernel(x, mask):
    return _masked_cumsum(x, mask.astype(jnp.float32))
